# Initial kernel scaffold; baseline (speedup 1.0000x reference)
import jax
import jax.numpy as jnp
from jax.experimental import pallas as pl


def kernel(x, edge_attr, edge_index, params):
    raise NotImplementedError("write your pallas kernel here")


# speedup vs baseline: 1.6504x; 1.6504x over previous
import functools

import jax
import jax.numpy as jnp
from jax import lax
from jax.experimental import pallas as pl
from jax.experimental.pallas import tpu as pltpu
from jax.experimental.pallas import tpu_sc as plsc

N = 10000
E = 160000
F = 128

NC = 2
NS = 16
NW = NC * NS


def _colstats_body(ea_ref, s_ref, q_ref):
    i = pl.program_id(0)

    @pl.when(i == 0)
    def _():
        s_ref[...] = jnp.zeros_like(s_ref)
        q_ref[...] = jnp.zeros_like(q_ref)

    blk = ea_ref[...]
    s_ref[...] += jnp.sum(blk, axis=0, keepdims=True)
    q_ref[...] += jnp.sum(blk * blk, axis=0, keepdims=True)


def _colstats(ea, blk=4000):
    return pl.pallas_call(
        _colstats_body,
        grid=(E // blk,),
        in_specs=[pl.BlockSpec((blk, F), lambda i: (i, 0))],
        out_specs=[pl.BlockSpec((1, F), lambda i: (0, 0)),
                   pl.BlockSpec((1, F), lambda i: (0, 0))],
        out_shape=[jax.ShapeDtypeStruct((1, F), jnp.float32),
                   jax.ShapeDtypeStruct((1, F), jnp.float32)],
    )(ea)


def _node_prep_body(x_ref, g_ref, b_ref, w1c_ref, w1r_ref,
                    h_ref, a_ref, bt_ref):
    x = x_ref[...]
    mu = jnp.mean(x, axis=0, keepdims=True)
    var = jnp.mean(x * x, axis=0, keepdims=True) - mu * mu
    hn = (x - mu) * jax.lax.rsqrt(var + 1e-5) * g_ref[...] + b_ref[...]
    h = jnp.maximum(hn, 0.0)
    h_ref[...] = h
    a_ref[...] = jnp.dot(h, w1c_ref[...], preferred_element_type=jnp.float32)
    bt_ref[...] = jnp.dot(h, w1r_ref[...], preferred_element_type=jnp.float32)


def _node_prep(x, g, b, w1c, w1r):
    return pl.pallas_call(
        _node_prep_body,
        out_shape=[jax.ShapeDtypeStruct((N, F), jnp.float32),
                   jax.ShapeDtypeStruct((N, 2 * F), jnp.float32),
                   jax.ShapeDtypeStruct((N, 2 * F), jnp.float32)],
    )(x, g, b, w1c, w1r)


def _edge_prep_body(ea_ref, s_ref, q_ref, g_ref, b_ref,
                    w1v_ref, b1_ref, ew1v_ref, eb1_ref, c_ref, qout_ref):
    mu = s_ref[...] * (1.0 / E)
    var = q_ref[...] * (1.0 / E) - mu * mu
    vn = (ea_ref[...] - mu) * jax.lax.rsqrt(var + 1e-5) * g_ref[...] + b_ref[...]
    v = jnp.maximum(vn, 0.0)
    c_ref[...] = jnp.dot(v, w1v_ref[...],
                         preferred_element_type=jnp.float32) + b1_ref[...]
    qout_ref[...] = jnp.dot(v, ew1v_ref[...],
                            preferred_element_type=jnp.float32) + eb1_ref[...]


def _edge_prep(ea, s, q, g, b, w1v, b1, ew1v, eb1, blk=2000):
    zero = lambda i: (0, 0)
    return pl.pallas_call(
        _edge_prep_body,
        grid=(E // blk,),
        in_specs=[pl.BlockSpec((blk, F), lambda i: (i, 0)),
                  pl.BlockSpec((1, F), zero), pl.BlockSpec((1, F), zero),
                  pl.BlockSpec((1, F), zero), pl.BlockSpec((1, F), zero),
                  pl.BlockSpec((F, 2 * F), zero), pl.BlockSpec((1, 2 * F), zero),
                  pl.BlockSpec((F, 2 * F), zero), pl.BlockSpec((1, 2 * F), zero)],
        out_specs=[pl.BlockSpec((blk, 2 * F), lambda i: (i, 0)),
                   pl.BlockSpec((blk, 2 * F), lambda i: (i, 0))],
        out_shape=[jax.ShapeDtypeStruct((E, 2 * F), jnp.float32),
                   jax.ShapeDtypeStruct((E, 2 * F), jnp.float32)],
    )(ea, s, q, g, b, w1v, b1, ew1v, eb1)


def _msg_body(pre_ref, w2_ref, b2_ref, t_ref, es_ref, w_ref):
    am = jnp.maximum(pre_ref[...], 0.0)
    m = jnp.dot(am, w2_ref[...], preferred_element_type=jnp.float32) + b2_ref[...]
    es = jnp.exp(m * t_ref[...])
    es_ref[...] = es
    w_ref[...] = m * es


def _msg_mm(pre, w2, b2, t, blk=2000):
    zero = lambda i: (0, 0)
    return pl.pallas_call(
        _msg_body,
        grid=(E // blk,),
        in_specs=[pl.BlockSpec((blk, 2 * F), lambda i: (i, 0)),
                  pl.BlockSpec((2 * F, F), zero), pl.BlockSpec((1, F), zero),
                  pl.BlockSpec((1, F), zero)],
        out_specs=[pl.BlockSpec((blk, F), lambda i: (i, 0)),
                   pl.BlockSpec((blk, F), lambda i: (i, 0))],
        out_shape=[jax.ShapeDtypeStruct((E, F), jnp.float32),
                   jax.ShapeDtypeStruct((E, F), jnp.float32)],
    )(pre, w2, b2, t)


def _node_mlp_body(h_ref, den_ref, num_ref,
                   w1_ref, b1_ref, g1_ref, bb1_ref,
                   w2_ref, b2_ref, g2_ref, bb2_ref,
                   w3_ref, b3_ref, g3_ref, bb3_ref,
                   w4_ref, b4_ref, hout_ref):
    den = den_ref[...]
    msg = jnp.where(den > 0.0, num_ref[...] / den, 0.0)
    hh = jnp.concatenate([h_ref[...], msg], axis=1)

    def layer(a, w, b, g, bb):
        y = jnp.dot(a, w[...], preferred_element_type=jnp.float32) + b[...]
        mu = jnp.mean(y, axis=0, keepdims=True)
        var = jnp.mean(y * y, axis=0, keepdims=True) - mu * mu
        yn = (y - mu) * jax.lax.rsqrt(var + 1e-5) * g[...] + bb[...]
        return jnp.maximum(yn, 0.0)

    hh = layer(hh, w1_ref, b1_ref, g1_ref, bb1_ref)
    hh = layer(hh, w2_ref, b2_ref, g2_ref, bb2_ref)
    hh = layer(hh, w3_ref, b3_ref, g3_ref, bb3_ref)
    hout_ref[...] = jnp.dot(hh, w4_ref[...],
                            preferred_element_type=jnp.float32) + b4_ref[...]


def _node_mlp(h, den, num, args):
    return pl.pallas_call(
        _node_mlp_body,
        out_shape=jax.ShapeDtypeStruct((N, F), jnp.float32),
    )(h, den, num, *args)


def _node_proj_body(hout_ref, wa_ref, wb_ref, pr_ref, pc_ref):
    r = jnp.maximum(hout_ref[...], 0.0)
    pr_ref[...] = jnp.dot(r, wa_ref[...], preferred_element_type=jnp.float32)
    pc_ref[...] = jnp.dot(r, wb_ref[...], preferred_element_type=jnp.float32)


def _node_proj(hout, wa, wb):
    return pl.pallas_call(
        _node_proj_body,
        out_shape=[jax.ShapeDtypeStruct((N, 2 * F), jnp.float32),
                   jax.ShapeDtypeStruct((N, 2 * F), jnp.float32)],
    )(hout, wa, wb)


def _vout_body(ev_ref, w_ref, b_ref, out_ref):
    ev = jnp.maximum(ev_ref[...], 0.0)
    out_ref[...] = jnp.dot(ev, w_ref[...],
                           preferred_element_type=jnp.float32) + b_ref[...]


def _vout_mm(ev, w, b, blk=2000):
    zero = lambda i: (0, 0)
    return pl.pallas_call(
        _vout_body,
        grid=(E // blk,),
        in_specs=[pl.BlockSpec((blk, 2 * F), lambda i: (i, 0)),
                  pl.BlockSpec((2 * F, F), zero), pl.BlockSpec((1, F), zero)],
        out_specs=pl.BlockSpec((blk, F), lambda i: (i, 0)),
        out_shape=jax.ShapeDtypeStruct((E, F), jnp.float32),
    )(ev, w, b)


_PER_W = E // NW
_GCH = 40


def _gather_add(a_tab, b_tab, c_arr, ia, ib):
    mesh = plsc.VectorSubcoreMesh(core_axis_name="c", subcore_axis_name="s")

    @functools.partial(
        pl.kernel, mesh=mesh,
        out_type=jax.ShapeDtypeStruct((E, 2 * F), jnp.float32),
        scratch_types=[
            pltpu.VMEM((_GCH,), jnp.int32),
            pltpu.VMEM((_GCH,), jnp.int32),
            pltpu.VMEM((_GCH, 2 * F), jnp.float32),
            pltpu.VMEM((_GCH, 2 * F), jnp.float32),
            pltpu.VMEM((_GCH, 2 * F), jnp.float32),
            pltpu.SemaphoreType.DMA,
            pltpu.SemaphoreType.DMA,
            pltpu.SemaphoreType.DMA,
        ],
    )
    def k(a_hbm, b_hbm, c_hbm, ia_hbm, ib_hbm, out_hbm,
          ia_v, ib_v, bufa, bufb, bufc, sma, smb, smc):
        wid = lax.axis_index("s") * NC + lax.axis_index("c")
        base0 = wid * _PER_W

        @pl.loop(0, _PER_W // _GCH)
        def _(i):
            base = base0 + i * _GCH
            pltpu.sync_copy(ia_hbm.at[pl.ds(base, _GCH)], ia_v)
            pltpu.sync_copy(ib_hbm.at[pl.ds(base, _GCH)], ib_v)
            cpa = pltpu.async_copy(a_hbm.at[ia_v], bufa, sma)
            cpb = pltpu.async_copy(b_hbm.at[ib_v], bufb, smb)
            cpc = pltpu.async_copy(c_hbm.at[pl.ds(base, _GCH), :], bufc, smc)
            cpa.wait()
            cpb.wait()
            cpc.wait()

            @pl.loop(0, _GCH)
            def _(r):
                @pl.loop(0, 2 * F, step=16)
                def _(f):
                    slc = (pl.ds(r, 1), pl.ds(f, 16))
                    bufa.at[*slc][...] = (bufa.at[*slc][...]
                                          + bufb.at[*slc][...]
                                          + bufc.at[*slc][...])

            pltpu.sync_copy(bufa, out_hbm.at[pl.ds(base, _GCH), :])

    return k(a_tab, b_tab, c_arr, ia, ib)


_PER_T = E // NS
_SCH = 80
_ZR = 200
_ZW = 10
_ROWS_PER_Z = N // _ZW


def _scatter_sum(es, w, col):
    mesh = plsc.VectorSubcoreMesh(core_axis_name="c", subcore_axis_name="s")

    @functools.partial(
        pl.kernel, mesh=mesh,
        out_type=jax.ShapeDtypeStruct((2 * N, F), jnp.float32),
        scratch_types=[
            pltpu.VMEM_SHARED((N, F), jnp.float32),
            pltpu.VMEM((_SCH,), jnp.int32),
            pltpu.VMEM((_SCH, F), jnp.float32),
            pltpu.VMEM((_ZR, F), jnp.float32),
        ],
    )
    def k(es_hbm, w_hbm, col_hbm, out_hbm, acc, idx_v, val_v, zbuf):
        cid = lax.axis_index("c")
        sid = lax.axis_index("s")

        @pl.when(sid < _ZW)
        def _():
            @pl.loop(0, _ZR)
            def _(r):
                @pl.loop(0, F, step=16)
                def _(f):
                    zbuf.at[pl.ds(r, 1), pl.ds(f, 16)][...] = jnp.zeros(
                        (1, 16), jnp.float32)

            @pl.loop(0, _ROWS_PER_Z // _ZR)
            def _(j):
                pltpu.sync_copy(
                    zbuf, acc.at[pl.ds(sid * _ROWS_PER_Z + j * _ZR, _ZR), :])

        plsc.subcore_barrier()

        def scat(src_hbm):
            @pl.loop(0, _PER_T // _SCH)
            def _(i):
                base = sid * _PER_T + i * _SCH
                pltpu.sync_copy(col_hbm.at[pl.ds(base, _SCH)], idx_v)
                pltpu.sync_copy(src_hbm.at[pl.ds(base, _SCH), :], val_v)
                pltpu.sync_copy(val_v, acc.at[idx_v], add=True)

        @pl.when(cid == 0)
        def _():
            scat(es_hbm)

        @pl.when(cid == 1)
        def _():
            scat(w_hbm)

        plsc.subcore_barrier()

        @pl.when(sid < _ZW)
        def _():
            rbase = sid * _ROWS_PER_Z
            pltpu.sync_copy(acc.at[pl.ds(rbase, _ROWS_PER_Z), :],
                            out_hbm.at[pl.ds(cid * N + rbase, _ROWS_PER_Z), :])

    return k(es, w, col)


def kernel(x, edge_attr, edge_index, params):
    p = params
    row = edge_index[0]
    col = edge_index[1]
    r1 = lambda a: a.reshape(1, -1)

    mp_w1 = p['mp_W1']
    em_w1 = p['em_W1']

    s, q = _colstats(edge_attr)
    h, a_tab, b_tab = _node_prep(x, r1(p['bn_h_g']), r1(p['bn_h_b']),
                                 mp_w1[:F], mp_w1[F:2 * F])
    c_arr, q_arr = _edge_prep(edge_attr, s, q, r1(p['bn_v_g']), r1(p['bn_v_b']),
                              mp_w1[2 * F:], r1(p['mp_b1']),
                              em_w1[2 * F:], r1(p['em_b1']))
    pre = _gather_add(a_tab, b_tab, c_arr, col, row)
    t_row = jnp.broadcast_to(p['t'].reshape(1, 1), (1, F))
    es, w = _msg_mm(pre, p['mp_W2'], r1(p['mp_b2']), t_row)
    dn = _scatter_sum(es, w, col)
    den, num = dn[:N], dn[N:]
    mlp_args = (p['mh_W1'], r1(p['mh_b1']), r1(p['mh_bn1_g']), r1(p['mh_bn1_b']),
                p['mh_W2'], r1(p['mh_b2']), r1(p['mh_bn2_g']), r1(p['mh_bn2_b']),
                p['mh_W3'], r1(p['mh_b3']), r1(p['mh_bn3_g']), r1(p['mh_bn3_b']),
                p['mh_W4'], r1(p['mh_b4']))
    h_out = _node_mlp(h, den, num, mlp_args)
    pr, pc = _node_proj(h_out, em_w1[:F], em_w1[F:2 * F])
    evpre = _gather_add(pr, pc, q_arr, row, col)
    v_out = _vout_mm(evpre, p['em_W2'], r1(p['em_b2']))
    return h_out, v_out


# speedup vs baseline: 3.6845x; 2.2325x over previous
import functools

import jax
import jax.numpy as jnp
from jax import lax
from jax.experimental import pallas as pl
from jax.experimental.pallas import tpu as pltpu
from jax.experimental.pallas import tpu_sc as plsc

N = 10000
E = 160000
F = 128

NC = 2
NS = 16
NW = NC * NS


def _colstats_body(ea_ref, s_ref, q_ref):
    i = pl.program_id(0)

    @pl.when(i == 0)
    def _():
        s_ref[...] = jnp.zeros_like(s_ref)
        q_ref[...] = jnp.zeros_like(q_ref)

    blk = ea_ref[...]
    s_ref[...] += jnp.sum(blk, axis=0, keepdims=True)
    q_ref[...] += jnp.sum(blk * blk, axis=0, keepdims=True)


def _colstats(ea, blk=4000):
    return pl.pallas_call(
        _colstats_body,
        grid=(E // blk,),
        in_specs=[pl.BlockSpec((blk, F), lambda i: (i, 0))],
        out_specs=[pl.BlockSpec((1, F), lambda i: (0, 0)),
                   pl.BlockSpec((1, F), lambda i: (0, 0))],
        out_shape=[jax.ShapeDtypeStruct((1, F), jnp.float32),
                   jax.ShapeDtypeStruct((1, F), jnp.float32)],
    )(ea)


def _node_prep_body(x_ref, g_ref, b_ref, w1c_ref, w1r_ref,
                    h_ref, a_ref, bt_ref):
    x = x_ref[...]
    mu = jnp.mean(x, axis=0, keepdims=True)
    var = jnp.mean(x * x, axis=0, keepdims=True) - mu * mu
    hn = (x - mu) * jax.lax.rsqrt(var + 1e-5) * g_ref[...] + b_ref[...]
    h = jnp.maximum(hn, 0.0)
    h_ref[...] = h
    a_ref[...] = jnp.dot(h, w1c_ref[...], preferred_element_type=jnp.float32)
    bt_ref[...] = jnp.dot(h, w1r_ref[...], preferred_element_type=jnp.float32)


def _node_prep(x, g, b, w1c, w1r):
    return pl.pallas_call(
        _node_prep_body,
        out_shape=[jax.ShapeDtypeStruct((N, F), jnp.float32),
                   jax.ShapeDtypeStruct((N, 2 * F), jnp.float32),
                   jax.ShapeDtypeStruct((N, 2 * F), jnp.float32)],
    )(x, g, b, w1c, w1r)


def _edge_prep_body(ea_ref, s_ref, q_ref, g_ref, b_ref,
                    w1v_ref, b1_ref, ew1v_ref, eb1_ref, c_ref, qout_ref):
    mu = s_ref[...] * (1.0 / E)
    var = q_ref[...] * (1.0 / E) - mu * mu
    vn = (ea_ref[...] - mu) * jax.lax.rsqrt(var + 1e-5) * g_ref[...] + b_ref[...]
    v = jnp.maximum(vn, 0.0)
    c_ref[...] = jnp.dot(v, w1v_ref[...],
                         preferred_element_type=jnp.float32) + b1_ref[...]
    qout_ref[...] = jnp.dot(v, ew1v_ref[...],
                            preferred_element_type=jnp.float32) + eb1_ref[...]


def _edge_prep(ea, s, q, g, b, w1v, b1, ew1v, eb1, blk=2000):
    zero = lambda i: (0, 0)
    return pl.pallas_call(
        _edge_prep_body,
        grid=(E // blk,),
        in_specs=[pl.BlockSpec((blk, F), lambda i: (i, 0)),
                  pl.BlockSpec((1, F), zero), pl.BlockSpec((1, F), zero),
                  pl.BlockSpec((1, F), zero), pl.BlockSpec((1, F), zero),
                  pl.BlockSpec((F, 2 * F), zero), pl.BlockSpec((1, 2 * F), zero),
                  pl.BlockSpec((F, 2 * F), zero), pl.BlockSpec((1, 2 * F), zero)],
        out_specs=[pl.BlockSpec((blk, 2 * F), lambda i: (i, 0)),
                   pl.BlockSpec((blk, 2 * F), lambda i: (i, 0))],
        out_shape=[jax.ShapeDtypeStruct((E, 2 * F), jnp.float32),
                   jax.ShapeDtypeStruct((E, 2 * F), jnp.float32)],
    )(ea, s, q, g, b, w1v, b1, ew1v, eb1)


def _msg_body(pre_ref, w2_ref, b2_ref, t_ref, es_ref, w_ref):
    am = jnp.maximum(pre_ref[...], 0.0)
    m = jnp.dot(am, w2_ref[...], preferred_element_type=jnp.float32) + b2_ref[...]
    es = jnp.exp(m * t_ref[...])
    es_ref[...] = es
    w_ref[...] = m * es


def _msg_mm(pre, w2, b2, t, blk=2000):
    zero = lambda i: (0, 0)
    return pl.pallas_call(
        _msg_body,
        grid=(E // blk,),
        in_specs=[pl.BlockSpec((blk, 2 * F), lambda i: (i, 0)),
                  pl.BlockSpec((2 * F, F), zero), pl.BlockSpec((1, F), zero),
                  pl.BlockSpec((1, F), zero)],
        out_specs=[pl.BlockSpec((blk, F), lambda i: (i, 0)),
                   pl.BlockSpec((blk, F), lambda i: (i, 0))],
        out_shape=[jax.ShapeDtypeStruct((E, F), jnp.float32),
                   jax.ShapeDtypeStruct((E, F), jnp.float32)],
    )(pre, w2, b2, t)


def _node_mlp_body(h_ref, den_ref, num_ref,
                   w1_ref, b1_ref, g1_ref, bb1_ref,
                   w2_ref, b2_ref, g2_ref, bb2_ref,
                   w3_ref, b3_ref, g3_ref, bb3_ref,
                   w4_ref, b4_ref, hout_ref):
    den = den_ref[...]
    msg = jnp.where(den > 0.0, num_ref[...] / den, 0.0)
    hh = jnp.concatenate([h_ref[...], msg], axis=1)

    def layer(a, w, b, g, bb):
        y = jnp.dot(a, w[...], preferred_element_type=jnp.float32) + b[...]
        mu = jnp.mean(y, axis=0, keepdims=True)
        var = jnp.mean(y * y, axis=0, keepdims=True) - mu * mu
        yn = (y - mu) * jax.lax.rsqrt(var + 1e-5) * g[...] + bb[...]
        return jnp.maximum(yn, 0.0)

    hh = layer(hh, w1_ref, b1_ref, g1_ref, bb1_ref)
    hh = layer(hh, w2_ref, b2_ref, g2_ref, bb2_ref)
    hh = layer(hh, w3_ref, b3_ref, g3_ref, bb3_ref)
    hout_ref[...] = jnp.dot(hh, w4_ref[...],
                            preferred_element_type=jnp.float32) + b4_ref[...]


def _node_mlp(h, den, num, args):
    return pl.pallas_call(
        _node_mlp_body,
        out_shape=jax.ShapeDtypeStruct((N, F), jnp.float32),
    )(h, den, num, *args)


def _node_proj_body(hout_ref, wa_ref, wb_ref, pr_ref, pc_ref):
    r = jnp.maximum(hout_ref[...], 0.0)
    pr_ref[...] = jnp.dot(r, wa_ref[...], preferred_element_type=jnp.float32)
    pc_ref[...] = jnp.dot(r, wb_ref[...], preferred_element_type=jnp.float32)


def _node_proj(hout, wa, wb):
    return pl.pallas_call(
        _node_proj_body,
        out_shape=[jax.ShapeDtypeStruct((N, 2 * F), jnp.float32),
                   jax.ShapeDtypeStruct((N, 2 * F), jnp.float32)],
    )(hout, wa, wb)


def _vout_body(ev_ref, w_ref, b_ref, out_ref):
    ev = jnp.maximum(ev_ref[...], 0.0)
    out_ref[...] = jnp.dot(ev, w_ref[...],
                           preferred_element_type=jnp.float32) + b_ref[...]


def _vout_mm(ev, w, b, blk=2000):
    zero = lambda i: (0, 0)
    return pl.pallas_call(
        _vout_body,
        grid=(E // blk,),
        in_specs=[pl.BlockSpec((blk, 2 * F), lambda i: (i, 0)),
                  pl.BlockSpec((2 * F, F), zero), pl.BlockSpec((1, F), zero)],
        out_specs=pl.BlockSpec((blk, F), lambda i: (i, 0)),
        out_shape=jax.ShapeDtypeStruct((E, F), jnp.float32),
    )(ev, w, b)


_PER_W = E // NW
_GCH = 40


_GK = _PER_W // _GCH


def _gather_add(a_tab, b_tab, c_arr, ia, ib):
    mesh = plsc.VectorSubcoreMesh(core_axis_name="c", subcore_axis_name="s")
    fbuf = pltpu.VMEM((_GCH, 2 * F), jnp.float32)

    @functools.partial(
        pl.kernel, mesh=mesh,
        out_type=jax.ShapeDtypeStruct((E, 2 * F), jnp.float32),
        scratch_types=(
            [pltpu.VMEM((_PER_W,), jnp.int32)] * 2
            + [fbuf] * 8
            + [pltpu.SemaphoreType.DMA] * 8
        ),
    )
    def k(a_hbm, b_hbm, c_hbm, ia_hbm, ib_hbm, out_hbm,
          ia_all, ib_all, ba0, ba1, bb0, bb1, bc0, bc1, bo0, bo1,
          sa0, sa1, sb0, sb1, sc0, sc1, so0, so1):
        wid = lax.axis_index("s") * NC + lax.axis_index("c")
        base0 = wid * _PER_W
        ba = (ba0, ba1)
        bb = (bb0, bb1)
        bc = (bc0, bc1)
        bo = (bo0, bo1)
        sa = (sa0, sa1)
        sb = (sb0, sb1)
        sc = (sc0, sc1)
        so = (so0, so1)

        pltpu.sync_copy(ia_hbm.at[pl.ds(base0, _PER_W)], ia_all)
        pltpu.sync_copy(ib_hbm.at[pl.ds(base0, _PER_W)], ib_all)

        def issue(kk, p):
            off = kk * _GCH
            base = base0 + off
            pltpu.async_copy(a_hbm.at[ia_all.at[pl.ds(off, _GCH)]], ba[p], sa[p])
            pltpu.async_copy(b_hbm.at[ib_all.at[pl.ds(off, _GCH)]], bb[p], sb[p])
            pltpu.async_copy(c_hbm.at[pl.ds(base, _GCH), :], bc[p], sc[p])

        def wait_in(p):
            src = c_hbm.at[pl.ds(0, _GCH), :]
            pltpu.make_async_copy(src, ba[p], sa[p]).wait()
            pltpu.make_async_copy(src, bb[p], sb[p]).wait()
            pltpu.make_async_copy(src, bc[p], sc[p]).wait()

        def wait_out(p):
            pltpu.make_async_copy(bo[p], out_hbm.at[pl.ds(0, _GCH), :],
                                  so[p]).wait()

        def compute(p):
            @pl.loop(0, _GCH)
            def _(r):
                for f in range(0, 2 * F, 16):
                    slc = (pl.ds(r, 1), pl.ds(f, 16))
                    bo[p].at[*slc][...] = (ba[p].at[*slc][...]
                                           + bb[p].at[*slc][...]
                                           + bc[p].at[*slc][...])

        def body(kk, p, first, last):
            wait_in(p)
            if not first:
                wait_out(p)
            compute(p)
            if not last:
                if isinstance(kk, int):
                    if kk + 2 < _GK:
                        issue(kk + 2, p)
                else:
                    @pl.when(kk + 2 < _GK)
                    def _():
                        issue(kk + 2, p)
            pltpu.async_copy(bo[p], out_hbm.at[pl.ds(base0 + kk * _GCH,
                                                     _GCH), :], so[p])

        issue(0, 0)
        issue(1, 1)
        body(0, 0, True, False)
        body(1, 1, True, False)

        @pl.loop(2, _GK - 1, step=2)
        def _(kk):
            body(kk, 0, False, False)
            body(kk + 1, 1, False, False)

        body(_GK - 1, 0, False, True)
        wait_out(1)
        wait_out(0)

    return k(a_tab, b_tab, c_arr, ia, ib)


_PER_T = E // NS
_SCH = 80
_ZR = 200
_ZW = 10
_ROWS_PER_Z = N // _ZW


def _scatter_sum(es, w, col):
    mesh = plsc.VectorSubcoreMesh(core_axis_name="c", subcore_axis_name="s")

    @functools.partial(
        pl.kernel, mesh=mesh,
        out_type=jax.ShapeDtypeStruct((2 * N, F), jnp.float32),
        scratch_types=[
            pltpu.VMEM_SHARED((N, F), jnp.float32),
            pltpu.VMEM((_SCH,), jnp.int32),
            pltpu.VMEM((_SCH,), jnp.int32),
            pltpu.VMEM((_SCH, F), jnp.float32),
            pltpu.VMEM((_SCH, F), jnp.float32),
            pltpu.VMEM((_ZR, F), jnp.float32),
            pltpu.SemaphoreType.DMA,
            pltpu.SemaphoreType.DMA,
            pltpu.SemaphoreType.DMA,
            pltpu.SemaphoreType.DMA,
        ],
    )
    def k(es_hbm, w_hbm, col_hbm, out_hbm, acc, idx_v0, idx_v1,
          val_v0, val_v1, zbuf, sl0, sl1, sc0, sc1):
        cid = lax.axis_index("c")
        sid = lax.axis_index("s")
        idx_v = (idx_v0, idx_v1)
        val_v = (val_v0, val_v1)
        sl = (sl0, sl1)
        ssc = (sc0, sc1)

        @pl.when(sid < _ZW)
        def _():
            @pl.loop(0, _ZR)
            def _(r):
                @pl.loop(0, F, step=16)
                def _(f):
                    zbuf.at[pl.ds(r, 1), pl.ds(f, 16)][...] = jnp.zeros(
                        (1, 16), jnp.float32)

            @pl.loop(0, _ROWS_PER_Z // _ZR)
            def _(j):
                pltpu.sync_copy(
                    zbuf, acc.at[pl.ds(sid * _ROWS_PER_Z + j * _ZR, _ZR), :])

        plsc.subcore_barrier()

        nk = _PER_T // _SCH

        def scat(src_hbm):
            def issue_loads(kk, p):
                base = sid * _PER_T + kk * _SCH
                pltpu.async_copy(col_hbm.at[pl.ds(base, _SCH)], idx_v[p], sl[p])
                pltpu.async_copy(src_hbm.at[pl.ds(base, _SCH), :], val_v[p],
                                 sl[p])

            def wait_loads(p):
                pltpu.make_async_copy(col_hbm.at[pl.ds(0, _SCH)], idx_v[p],
                                      sl[p]).wait()
                pltpu.make_async_copy(src_hbm.at[pl.ds(0, _SCH), :], val_v[p],
                                      sl[p]).wait()

            def wait_scat(p):
                pltpu.make_async_copy(val_v[p], acc.at[idx_v[p]],
                                      ssc[p]).wait()

            def body(kk, p, first):
                wait_loads(p)
                q = 1 - p
                if not first:
                    wait_scat(q)
                if isinstance(kk, int):
                    if kk + 1 < nk:
                        issue_loads(kk + 1, q)
                else:
                    @pl.when(kk + 1 < nk)
                    def _():
                        issue_loads(kk + 1, q)
                pltpu.async_copy(val_v[p], acc.at[idx_v[p]], ssc[p],
                                 add=True)

            issue_loads(0, 0)
            body(0, 0, True)

            @pl.loop(1, nk - 1, step=2)
            def _(kk):
                body(kk, 1, False)
                body(kk + 1, 0, False)

            wait_scat(0)

        @pl.when(cid == 0)
        def _():
            scat(es_hbm)

        @pl.when(cid == 1)
        def _():
            scat(w_hbm)

        plsc.subcore_barrier()

        @pl.when(sid < _ZW)
        def _():
            rbase = sid * _ROWS_PER_Z
            pltpu.sync_copy(acc.at[pl.ds(rbase, _ROWS_PER_Z), :],
                            out_hbm.at[pl.ds(cid * N + rbase, _ROWS_PER_Z), :])

    return k(es, w, col)


def kernel(x, edge_attr, edge_index, params):
    p = params
    row = edge_index[0]
    col = edge_index[1]
    r1 = lambda a: a.reshape(1, -1)

    mp_w1 = p['mp_W1']
    em_w1 = p['em_W1']

    s, q = _colstats(edge_attr)
    h, a_tab, b_tab = _node_prep(x, r1(p['bn_h_g']), r1(p['bn_h_b']),
                                 mp_w1[:F], mp_w1[F:2 * F])
    c_arr, q_arr = _edge_prep(edge_attr, s, q, r1(p['bn_v_g']), r1(p['bn_v_b']),
                              mp_w1[2 * F:], r1(p['mp_b1']),
                              em_w1[2 * F:], r1(p['em_b1']))
    pre = _gather_add(a_tab, b_tab, c_arr, col, row)
    t_row = jnp.broadcast_to(p['t'].reshape(1, 1), (1, F))
    es, w = _msg_mm(pre, p['mp_W2'], r1(p['mp_b2']), t_row)
    dn = _scatter_sum(es, w, col)
    den, num = dn[:N], dn[N:]
    mlp_args = (p['mh_W1'], r1(p['mh_b1']), r1(p['mh_bn1_g']), r1(p['mh_bn1_b']),
                p['mh_W2'], r1(p['mh_b2']), r1(p['mh_bn2_g']), r1(p['mh_bn2_b']),
                p['mh_W3'], r1(p['mh_b3']), r1(p['mh_bn3_g']), r1(p['mh_bn3_b']),
                p['mh_W4'], r1(p['mh_b4']))
    h_out = _node_mlp(h, den, num, mlp_args)
    pr, pc = _node_proj(h_out, em_w1[:F], em_w1[F:2 * F])
    evpre = _gather_add(pr, pc, q_arr, row, col)
    v_out = _vout_mm(evpre, p['em_W2'], r1(p['em_b2']))
    return h_out, v_out


# speedup vs baseline: 4.7084x; 1.2779x over previous
import dataclasses
import functools

import jax
import jax.numpy as jnp
from jax import lax
from jax.experimental import pallas as pl
from jax.experimental.pallas import tpu as pltpu
from jax.experimental.pallas import tpu_sc as plsc

N = 10000
E = 160000
F = 128

NC = 2
NS = 16
NW = NC * NS

BF = jnp.bfloat16

_SC_CP = pltpu.CompilerParams()
if "needs_layout_passes" in pltpu.CompilerParams.__dataclass_fields__:
    _SC_CP = dataclasses.replace(_SC_CP, needs_layout_passes=False)


def _pack_bf16(x):
    k = x.shape[1] // 2
    u = jax.lax.bitcast_convert_type(x, jnp.uint32)
    b = (u + (((u >> 16) & 1) + 0x7FFF)) >> 16
    return b[:, :k] | (b[:, k:] << 16)


def _unpack_bf16(u):
    lo = jax.lax.bitcast_convert_type(u << 16, jnp.float32)
    hi = jax.lax.bitcast_convert_type(u & jnp.uint32(0xFFFF0000), jnp.float32)
    return jnp.concatenate([lo, hi], axis=1)


def _colstats_body(ea_ref, s_ref, q_ref):
    i = pl.program_id(0)

    @pl.when(i == 0)
    def _():
        s_ref[...] = jnp.zeros_like(s_ref)
        q_ref[...] = jnp.zeros_like(q_ref)

    blk = ea_ref[...].reshape(-1, 8, F)
    s_ref[...] += jnp.sum(blk, axis=0)
    q_ref[...] += jnp.sum(blk * blk, axis=0)


def _colstats(ea, blk=4000):
    return pl.pallas_call(
        _colstats_body,
        grid=(E // blk,),
        in_specs=[pl.BlockSpec((blk, F), lambda i: (i, 0))],
        out_specs=[pl.BlockSpec((8, F), lambda i: (0, 0)),
                   pl.BlockSpec((8, F), lambda i: (0, 0))],
        out_shape=[jax.ShapeDtypeStruct((8, F), jnp.float32),
                   jax.ShapeDtypeStruct((8, F), jnp.float32)],
    )(ea)


def _node_prep_body(x_ref, g_ref, b_ref, w1c_ref, w1r_ref,
                    h_ref, a_ref, bt_ref):
    x = x_ref[...]
    mu = jnp.mean(x, axis=0, keepdims=True)
    var = jnp.mean(x * x, axis=0, keepdims=True) - mu * mu
    hn = (x - mu) * jax.lax.rsqrt(var + 1e-5) * g_ref[...] + b_ref[...]
    h = jnp.maximum(hn, 0.0)
    h_ref[...] = h
    hb = h.astype(BF)
    a = jnp.dot(hb, w1c_ref[...], preferred_element_type=jnp.float32)
    bt = jnp.dot(hb, w1r_ref[...], preferred_element_type=jnp.float32)
    a_ref[...] = _pack_bf16(a)
    bt_ref[...] = _pack_bf16(bt)


def _node_prep(x, g, b, w1c, w1r):
    tab = jax.ShapeDtypeStruct((N, F), jnp.uint32)
    return pl.pallas_call(
        _node_prep_body,
        out_shape=[jax.ShapeDtypeStruct((N, F), jnp.float32), tab, tab],
    )(x, g, b, w1c, w1r)


def _edge_prep_body(ea_ref, s_ref, q_ref, g_ref, b_ref,
                    w1v_ref, b1_ref, ew1v_ref, eb1_ref, c_ref, qq_ref):
    mu = jnp.sum(s_ref[...], axis=0, keepdims=True) * (1.0 / E)
    m2 = jnp.sum(q_ref[...], axis=0, keepdims=True) * (1.0 / E)
    var = m2 - mu * mu
    vn = (ea_ref[...] - mu) * jax.lax.rsqrt(var + 1e-5) * g_ref[...] + b_ref[...]
    v = jnp.maximum(vn, 0.0).astype(BF)
    c = jnp.dot(v, w1v_ref[...], preferred_element_type=jnp.float32) + b1_ref[...]
    qq = jnp.dot(v, ew1v_ref[...], preferred_element_type=jnp.float32) + eb1_ref[...]
    c_ref[...] = _pack_bf16(c)
    qq_ref[...] = _pack_bf16(qq)


def _edge_prep(ea, s, q, g, b, w1v, b1, ew1v, eb1, blk=2000):
    zero = lambda i: (0, 0)
    obs = pl.BlockSpec((blk, F), lambda i: (i, 0))
    osh = jax.ShapeDtypeStruct((E, F), jnp.uint32)
    return pl.pallas_call(
        _edge_prep_body,
        grid=(E // blk,),
        in_specs=[pl.BlockSpec((blk, F), lambda i: (i, 0)),
                  pl.BlockSpec((8, F), zero), pl.BlockSpec((8, F), zero),
                  pl.BlockSpec((1, F), zero), pl.BlockSpec((1, F), zero),
                  pl.BlockSpec((F, 2 * F), zero), pl.BlockSpec((1, 2 * F), zero),
                  pl.BlockSpec((F, 2 * F), zero), pl.BlockSpec((1, 2 * F), zero)],
        out_specs=[obs, obs],
        out_shape=[osh, osh],
    )(ea, s, q, g, b, w1v, b1, ew1v, eb1)


def _msg_body(pk_ref, w2_ref, b2_ref, t_ref, es_ref, w_ref):
    pre = _unpack_bf16(pk_ref[...])
    am = jnp.maximum(pre, 0.0).astype(BF)
    m = jnp.dot(am, w2_ref[...], preferred_element_type=jnp.float32) + b2_ref[...]
    es = jnp.exp(m * t_ref[...])
    es_ref[...] = es
    w_ref[...] = m * es


def _msg_mm(pk, w2, b2, t, blk=2000):
    zero = lambda i: (0, 0)
    return pl.pallas_call(
        _msg_body,
        grid=(E // blk,),
        in_specs=[pl.BlockSpec((blk, F), lambda i: (i, 0)),
                  pl.BlockSpec((2 * F, F), zero), pl.BlockSpec((1, F), zero),
                  pl.BlockSpec((1, F), zero)],
        out_specs=[pl.BlockSpec((blk, F), lambda i: (i, 0)),
                   pl.BlockSpec((blk, F), lambda i: (i, 0))],
        out_shape=[jax.ShapeDtypeStruct((E, F), jnp.float32),
                   jax.ShapeDtypeStruct((E, F), jnp.float32)],
    )(pk, w2, b2, t)


def _node_mlp_body(h_ref, dn_ref,
                   w1_ref, b1_ref, g1_ref, bb1_ref,
                   w2_ref, b2_ref, g2_ref, bb2_ref,
                   w3_ref, b3_ref, g3_ref, bb3_ref,
                   w4_ref, b4_ref, hout_ref):
    den = dn_ref[0:N, :]
    num = dn_ref[N:2 * N, :]
    msg = jnp.where(den > 0.0, num / den, 0.0)
    hh = jnp.concatenate([h_ref[...], msg], axis=1)

    def layer(a, w, b, g, bb):
        y = jnp.dot(a, w[...], preferred_element_type=jnp.float32) + b[...]
        mu = jnp.mean(y, axis=0, keepdims=True)
        var = jnp.mean(y * y, axis=0, keepdims=True) - mu * mu
        yn = (y - mu) * jax.lax.rsqrt(var + 1e-5) * g[...] + bb[...]
        return jnp.maximum(yn, 0.0)

    hh = layer(hh, w1_ref, b1_ref, g1_ref, bb1_ref)
    hh = layer(hh, w2_ref, b2_ref, g2_ref, bb2_ref)
    hh = layer(hh, w3_ref, b3_ref, g3_ref, bb3_ref)
    hout_ref[...] = jnp.dot(hh, w4_ref[...],
                            preferred_element_type=jnp.float32) + b4_ref[...]


def _node_mlp(h, dn, args):
    return pl.pallas_call(
        _node_mlp_body,
        out_shape=jax.ShapeDtypeStruct((N, F), jnp.float32),
    )(h, dn, *args)


def _node_proj_body(hout_ref, wa_ref, wb_ref, pr_ref, pc_ref):
    r = jnp.maximum(hout_ref[...], 0.0).astype(BF)
    pr = jnp.dot(r, wa_ref[...], preferred_element_type=jnp.float32)
    pc = jnp.dot(r, wb_ref[...], preferred_element_type=jnp.float32)
    pr_ref[...] = _pack_bf16(pr)
    pc_ref[...] = _pack_bf16(pc)


def _node_proj(hout, wa, wb):
    tab = jax.ShapeDtypeStruct((N, F), jnp.uint32)
    return pl.pallas_call(
        _node_proj_body,
        out_shape=[tab, tab],
    )(hout, wa, wb)


def _vout_body(ek_ref, w_ref, b_ref, out_ref):
    pre = _unpack_bf16(ek_ref[...])
    ev = jnp.maximum(pre, 0.0).astype(BF)
    out_ref[...] = jnp.dot(ev, w_ref[...],
                           preferred_element_type=jnp.float32) + b_ref[...]


def _vout_mm(ek, w, b, blk=2000):
    zero = lambda i: (0, 0)
    return pl.pallas_call(
        _vout_body,
        grid=(E // blk,),
        in_specs=[pl.BlockSpec((blk, F), lambda i: (i, 0)),
                  pl.BlockSpec((2 * F, F), zero), pl.BlockSpec((1, F), zero)],
        out_specs=pl.BlockSpec((blk, F), lambda i: (i, 0)),
        out_shape=jax.ShapeDtypeStruct((E, F), jnp.float32),
    )(ek, w, b)


_PER_W = E // NW
_GCH = 40
_GK = _PER_W // _GCH


def _gather_add(a_tab, b_tab, c_arr, ia, ib):
    mesh = plsc.VectorSubcoreMesh(core_axis_name="c", subcore_axis_name="s")
    hbuf = pltpu.VMEM((_GCH, F), jnp.uint32)

    @functools.partial(
        pl.kernel, mesh=mesh,
        out_type=jax.ShapeDtypeStruct((E, F), jnp.uint32),
        compiler_params=_SC_CP,
        scratch_types=(
            [pltpu.VMEM((_PER_W,), jnp.int32)] * 2
            + [hbuf] * 8
            + [pltpu.SemaphoreType.DMA] * 8
        ),
    )
    def k(a_hbm, b_hbm, c_hbm, ia_hbm, ib_hbm, out_hbm,
          ia_all, ib_all, ba0, ba1, bb0, bb1, bc0, bc1, bo0, bo1,
          sa0, sa1, sb0, sb1, sc0, sc1, so0, so1):
        wid = lax.axis_index("s") * NC + lax.axis_index("c")
        base0 = wid * _PER_W
        ba = (ba0, ba1)
        bb = (bb0, bb1)
        bc = (bc0, bc1)
        bo = (bo0, bo1)
        sa = (sa0, sa1)
        sb = (sb0, sb1)
        sc_ = (sc0, sc1)
        so = (so0, so1)

        pltpu.sync_copy(ia_hbm.at[pl.ds(base0, _PER_W)], ia_all)
        pltpu.sync_copy(ib_hbm.at[pl.ds(base0, _PER_W)], ib_all)

        def issue(kk, p):
            off = kk * _GCH
            pltpu.async_copy(a_hbm.at[ia_all.at[pl.ds(off, _GCH)]], ba[p], sa[p])
            pltpu.async_copy(b_hbm.at[ib_all.at[pl.ds(off, _GCH)]], bb[p], sb[p])
            pltpu.async_copy(c_hbm.at[pl.ds(base0 + off, _GCH), :], bc[p], sc_[p])

        def wait_in(p):
            src = c_hbm.at[pl.ds(0, _GCH), :]
            pltpu.make_async_copy(src, ba[p], sa[p]).wait()
            pltpu.make_async_copy(src, bb[p], sb[p]).wait()
            pltpu.make_async_copy(src, bc[p], sc_[p]).wait()

        def wait_out(p):
            pltpu.make_async_copy(bo[p], out_hbm.at[pl.ds(0, _GCH), :],
                                  so[p]).wait()

        def compute(p):
            @pl.loop(0, _GCH)
            def _(r):
                for f in range(0, F, 16):
                    slc = (r, pl.ds(f, 16))
                    av = plsc.bitcast(ba[p].at[*slc][...], BF)
                    bv = plsc.bitcast(bb[p].at[*slc][...], BF)
                    cv = plsc.bitcast(bc[p].at[*slc][...], BF)
                    bo[p].at[*slc][...] = plsc.bitcast(av + bv + cv,
                                                       jnp.uint32)

        def body(kk, p, first, last):
            wait_in(p)
            if not first:
                wait_out(p)
            compute(p)
            if not last:
                if isinstance(kk, int):
                    if kk + 2 < _GK:
                        issue(kk + 2, p)
                else:
                    @pl.when(kk + 2 < _GK)
                    def _():
                        issue(kk + 2, p)
            pltpu.async_copy(bo[p], out_hbm.at[pl.ds(base0 + kk * _GCH,
                                                     _GCH), :], so[p])

        issue(0, 0)
        issue(1, 1)
        body(0, 0, True, False)
        body(1, 1, True, False)

        @pl.loop(2, _GK - 1, step=2)
        def _(kk):
            body(kk, 0, False, False)
            body(kk + 1, 1, False, False)

        body(_GK - 1, 0, False, True)
        wait_out(1)
        wait_out(0)

    return k(a_tab, b_tab, c_arr, ia, ib)


_PER_T = E // NS
_SCH = 80
_ZR = 200
_ZW = 10
_ROWS_PER_Z = N // _ZW


def _scatter_sum(es, w, col):
    mesh = plsc.VectorSubcoreMesh(core_axis_name="c", subcore_axis_name="s")

    @functools.partial(
        pl.kernel, mesh=mesh,
        out_type=jax.ShapeDtypeStruct((2 * N, F), jnp.float32),
        scratch_types=[
            pltpu.VMEM_SHARED((N, F), jnp.float32),
            pltpu.VMEM((_SCH,), jnp.int32),
            pltpu.VMEM((_SCH,), jnp.int32),
            pltpu.VMEM((_SCH, F), jnp.float32),
            pltpu.VMEM((_SCH, F), jnp.float32),
            pltpu.VMEM((_ZR, F), jnp.float32),
            pltpu.SemaphoreType.DMA,
            pltpu.SemaphoreType.DMA,
            pltpu.SemaphoreType.DMA,
            pltpu.SemaphoreType.DMA,
        ],
    )
    def k(es_hbm, w_hbm, col_hbm, out_hbm, acc, idx_v0, idx_v1,
          val_v0, val_v1, zbuf, sl0, sl1, sc0, sc1):
        cid = lax.axis_index("c")
        sid = lax.axis_index("s")
        idx_v = (idx_v0, idx_v1)
        val_v = (val_v0, val_v1)
        sl = (sl0, sl1)
        ssc = (sc0, sc1)

        @pl.when(sid < _ZW)
        def _():
            @pl.loop(0, _ZR)
            def _(r):
                @pl.loop(0, F, step=16)
                def _(f):
                    zbuf.at[pl.ds(r, 1), pl.ds(f, 16)][...] = jnp.zeros(
                        (1, 16), jnp.float32)

            @pl.loop(0, _ROWS_PER_Z // _ZR)
            def _(j):
                pltpu.sync_copy(
                    zbuf, acc.at[pl.ds(sid * _ROWS_PER_Z + j * _ZR, _ZR), :])

        plsc.subcore_barrier()

        nk = _PER_T // _SCH

        def scat(src_hbm):
            def issue_loads(kk, p):
                base = sid * _PER_T + kk * _SCH
                pltpu.async_copy(col_hbm.at[pl.ds(base, _SCH)], idx_v[p], sl[p])
                pltpu.async_copy(src_hbm.at[pl.ds(base, _SCH), :], val_v[p],
                                 sl[p])

            def wait_loads(p):
                pltpu.make_async_copy(col_hbm.at[pl.ds(0, _SCH)], idx_v[p],
                                      sl[p]).wait()
                pltpu.make_async_copy(src_hbm.at[pl.ds(0, _SCH), :], val_v[p],
                                      sl[p]).wait()

            def wait_scat(p):
                pltpu.make_async_copy(val_v[p], acc.at[idx_v[p]],
                                      ssc[p]).wait()

            def body(kk, p, first):
                wait_loads(p)
                q = 1 - p
                if not first:
                    wait_scat(q)
                if isinstance(kk, int):
                    if kk + 1 < nk:
                        issue_loads(kk + 1, q)
                else:
                    @pl.when(kk + 1 < nk)
                    def _():
                        issue_loads(kk + 1, q)
                pltpu.async_copy(val_v[p], acc.at[idx_v[p]], ssc[p],
                                 add=True)

            issue_loads(0, 0)
            body(0, 0, True)

            @pl.loop(1, nk - 1, step=2)
            def _(kk):
                body(kk, 1, False)
                body(kk + 1, 0, False)

            wait_scat(0)

        @pl.when(cid == 0)
        def _():
            scat(es_hbm)

        @pl.when(cid == 1)
        def _():
            scat(w_hbm)

        plsc.subcore_barrier()

        @pl.when(sid < _ZW)
        def _():
            rbase = sid * _ROWS_PER_Z
            pltpu.sync_copy(acc.at[pl.ds(rbase, _ROWS_PER_Z), :],
                            out_hbm.at[pl.ds(cid * N + rbase, _ROWS_PER_Z), :])

    return k(es, w, col)


def kernel(x, edge_attr, edge_index, params):
    p = params
    row = edge_index[0]
    col = edge_index[1]
    r1 = lambda a: a.reshape(1, -1)

    mp_w1 = p['mp_W1']
    em_w1 = p['em_W1']

    s, q = _colstats(edge_attr)
    h, a_tab, b_tab = _node_prep(x, r1(p['bn_h_g']), r1(p['bn_h_b']),
                                 mp_w1[:F].astype(BF),
                                 mp_w1[F:2 * F].astype(BF))
    c_pk, q_pk = _edge_prep(edge_attr, s, q,
                            r1(p['bn_v_g']), r1(p['bn_v_b']),
                            mp_w1[2 * F:].astype(BF), r1(p['mp_b1']),
                            em_w1[2 * F:].astype(BF), r1(p['em_b1']))
    pre_pk = _gather_add(a_tab, b_tab, c_pk, col, row)
    t_row = jnp.broadcast_to(p['t'].reshape(1, 1), (1, F))
    es, w = _msg_mm(pre_pk, p['mp_W2'].astype(BF), r1(p['mp_b2']), t_row)
    dn = _scatter_sum(es, w, col)
    mlp_args = (p['mh_W1'], r1(p['mh_b1']), r1(p['mh_bn1_g']), r1(p['mh_bn1_b']),
                p['mh_W2'], r1(p['mh_b2']), r1(p['mh_bn2_g']), r1(p['mh_bn2_b']),
                p['mh_W3'], r1(p['mh_b3']), r1(p['mh_bn3_g']), r1(p['mh_bn3_b']),
                p['mh_W4'], r1(p['mh_b4']))
    h_out = _node_mlp(h, dn, mlp_args)
    pr_pk, pc_pk = _node_proj(h_out, em_w1[:F].astype(BF),
                              em_w1[F:2 * F].astype(BF))
    ev_pk = _gather_add(pr_pk, pc_pk, q_pk, row, col)
    v_out = _vout_mm(ev_pk, p['em_W2'].astype(BF), r1(p['em_b2']))
    return h_out, v_out
